# Initial kernel scaffold; baseline (speedup 1.0000x reference)
import functools

import jax
import jax.numpy as jnp
from jax import lax
from jax.experimental import pallas as pl
from jax.experimental.pallas import tpu as pltpu
from jax.experimental.pallas import tpu_sc as plsc

_R = 32
_NVOX = _R ** 3
_B = 16
_C = 64
_N = 32768
_L = 16
_CHUNK = 8192


def _norm_body(coords_ref, nc_ref, flat_ref):
    c = coords_ref[0]
    mean = jnp.mean(c, axis=1, keepdims=True)
    cen = c - mean
    sq = jnp.sum(cen * cen, axis=0, keepdims=True)
    denom = jnp.sqrt(jnp.max(sq)) * 2.0
    nc = cen / denom + 0.5
    nc = jnp.clip(nc * float(_R), 0.0, float(_R - 1))
    nc_ref[0] = nc
    vox = jnp.round(nc).astype(jnp.int32)
    flat_ref[0, 0] = (vox[0] * _R + vox[1]) * _R + vox[2]


def _normalize(coords):
    return pl.pallas_call(
        _norm_body,
        grid=(_B,),
        in_specs=[pl.BlockSpec((1, 3, _N), lambda b: (b, 0, 0))],
        out_specs=[
            pl.BlockSpec((1, 3, _N), lambda b: (b, 0, 0)),
            pl.BlockSpec((1, 1, _N), lambda b: (b, 0, 0)),
        ],
        out_shape=[
            jax.ShapeDtypeStruct((_B, 3, _N), jnp.float32),
            jax.ShapeDtypeStruct((_B, 1, _N), jnp.int32),
        ],
    )(coords)


def _scatter_body(feat_hbm, flat_hbm, out_hbm, idx_v, acc_v, cnt_v, fbuf):
    cid = lax.axis_index("c")
    sid = lax.axis_index("s")
    wid = sid * 2 + cid
    b = wid // 2
    c0 = (wid % 2) * (_C // 2)

    pltpu.sync_copy(flat_hbm.at[b], idx_v)

    zeros = jnp.zeros((_L,), jnp.float32)
    ones = jnp.ones((_L,), jnp.float32)

    def _zero_cnt(i, carry):
        cnt_v[pl.ds(i * _L, _L)] = zeros
        return carry
    lax.fori_loop(0, _NVOX // _L, _zero_cnt, 0)

    def _count(i, carry):
        iv = idx_v[pl.ds(i * _L, _L)]
        plsc.addupdate_scatter(cnt_v, [iv], ones)
        return carry
    lax.fori_loop(0, _N // _L, _count, 0)

    def _recip(i, carry):
        s = pl.ds(i * _L, _L)
        cnt_v[s] = 1.0 / jnp.maximum(cnt_v[s], 1.0)
        return carry
    lax.fori_loop(0, _NVOX // _L, _recip, 0)

    def _channel(c, carry):
        def _zero_acc(i, inner):
            acc_v[pl.ds(i * _L, _L)] = zeros
            return inner
        lax.fori_loop(0, _NVOX // _L, _zero_acc, 0)

        def _chunk(k, inner):
            pltpu.sync_copy(
                feat_hbm.at[b, c0 + c, pl.ds(k * _CHUNK, _CHUNK)], fbuf)

            def _scat(j, inner2):
                iv = idx_v[pl.ds(k * _CHUNK + j * _L, _L)]
                fv = fbuf[pl.ds(j * _L, _L)]
                plsc.addupdate_scatter(acc_v, [iv], fv)
                return inner2
            lax.fori_loop(0, _CHUNK // _L, _scat, 0)
            return inner
        lax.fori_loop(0, _N // _CHUNK, _chunk, 0)

        def _fin(i, inner):
            s = pl.ds(i * _L, _L)
            acc_v[s] = acc_v[s] * cnt_v[s]
            return inner
        lax.fori_loop(0, _NVOX // _L, _fin, 0)

        pltpu.sync_copy(acc_v, out_hbm.at[b, c0 + c])
        return carry
    lax.fori_loop(0, _C // 2, _channel, 0)


def _scatter(features, flat):
    mesh = plsc.VectorSubcoreMesh(core_axis_name="c", subcore_axis_name="s")
    return pl.kernel(
        _scatter_body,
        out_type=jax.ShapeDtypeStruct((_B, _C, _NVOX), jnp.float32),
        mesh=mesh,
        scratch_types=[
            pltpu.VMEM((_N,), jnp.int32),
            pltpu.VMEM((_NVOX,), jnp.float32),
            pltpu.VMEM((_NVOX,), jnp.float32),
            pltpu.VMEM((_CHUNK,), jnp.float32),
        ],
    )(features, flat)


def kernel(features, coords):
    norm_coords, flat = _normalize(coords)
    flat2 = flat.reshape(_B, _N)
    vox = _scatter(features, flat2)
    return vox.reshape(_B, _C, _R, _R, _R), norm_coords


# speedup vs baseline: 1.6226x; 1.6226x over previous
import functools

import jax
import jax.numpy as jnp
from jax import lax
from jax.experimental import pallas as pl
from jax.experimental.pallas import tpu as pltpu
from jax.experimental.pallas import tpu_sc as plsc

_R = 32
_NVOX = _R ** 3
_B = 16
_C = 64
_N = 32768
_L = 16
_CHUNK = 8192


def _norm_body(coords_ref, nc_ref, flat_ref):
    c = coords_ref[0]
    mean = jnp.mean(c, axis=1, keepdims=True)
    cen = c - mean
    sq = jnp.sum(cen * cen, axis=0, keepdims=True)
    denom = jnp.sqrt(jnp.max(sq)) * 2.0
    nc = cen / denom + 0.5
    nc = jnp.clip(nc * float(_R), 0.0, float(_R - 1))
    nc_ref[0] = nc
    vox = jnp.round(nc).astype(jnp.int32)
    flat_ref[0, 0] = (vox[0] * _R + vox[1]) * _R + vox[2]


def _normalize(coords):
    return pl.pallas_call(
        _norm_body,
        grid=(_B,),
        in_specs=[pl.BlockSpec((1, 3, _N), lambda b: (b, 0, 0))],
        out_specs=[
            pl.BlockSpec((1, 3, _N), lambda b: (b, 0, 0)),
            pl.BlockSpec((1, 1, _N), lambda b: (b, 0, 0)),
        ],
        out_shape=[
            jax.ShapeDtypeStruct((_B, 3, _N), jnp.float32),
            jax.ShapeDtypeStruct((_B, 1, _N), jnp.int32),
        ],
    )(coords)


def _scatter_body(feat_hbm, flat_hbm, out_hbm, idx_v, acc_v, cnt_v, fbuf):
    cid = lax.axis_index("c")
    sid = lax.axis_index("s")
    wid = sid * 2 + cid
    b = wid // 2
    c0 = (wid % 2) * (_C // 2)

    pltpu.sync_copy(flat_hbm.at[b], idx_v)

    zeros = jnp.zeros((_L,), jnp.float32)
    ones = jnp.ones((_L,), jnp.float32)

    def _zero_cnt(i, carry):
        cnt_v[pl.ds(i * _L, _L)] = zeros
        return carry
    lax.fori_loop(0, _NVOX // _L, _zero_cnt, 0)

    def _count(i, carry):
        iv = idx_v[pl.ds(i * _L, _L)]
        plsc.addupdate_scatter(cnt_v, [iv], ones)
        return carry
    lax.fori_loop(0, _N // _L, _count, 0)

    def _recip(i, carry):
        s = pl.ds(i * _L, _L)
        cnt_v[s] = 1.0 / jnp.maximum(cnt_v[s], 1.0)
        return carry
    lax.fori_loop(0, _NVOX // _L, _recip, 0)

    def _channel(c, carry):
        def _zero_acc(i, inner):
            acc_v[pl.ds(i * _L, _L)] = zeros
            return inner
        lax.fori_loop(0, _NVOX // _L, _zero_acc, 0)

        def _chunk(k, inner):
            pltpu.sync_copy(
                feat_hbm.at[b, c0 + c, pl.ds(k * _CHUNK, _CHUNK)], fbuf)

            def _scat(j, inner2):
                iv = idx_v[pl.ds(k * _CHUNK + j * _L, _L)]
                fv = fbuf[pl.ds(j * _L, _L)]
                plsc.addupdate_scatter(acc_v, [iv], fv)
                return inner2
            lax.fori_loop(0, _CHUNK // _L, _scat, 0)
            return inner
        lax.fori_loop(0, _N // _CHUNK, _chunk, 0)

        def _fin(i, inner):
            s = pl.ds(i * _L, _L)
            acc_v[s] = acc_v[s] * cnt_v[s]
            return inner
        lax.fori_loop(0, _NVOX // _L, _fin, 0)

        pltpu.sync_copy(acc_v, out_hbm.at[b, c0 + c])
        return carry
    lax.fori_loop(0, _C // 2, _channel, 0)


def _scatter(features, flat):
    mesh = plsc.VectorSubcoreMesh(core_axis_name="c", subcore_axis_name="s")
    return pl.kernel(
        _scatter_body,
        out_type=jax.ShapeDtypeStruct((_B, _C, _NVOX), jnp.float32),
        mesh=mesh,
        compiler_params=pltpu.CompilerParams(needs_layout_passes=False),
        scratch_types=[
            pltpu.VMEM((_N,), jnp.int32),
            pltpu.VMEM((_NVOX,), jnp.float32),
            pltpu.VMEM((_NVOX,), jnp.float32),
            pltpu.VMEM((_CHUNK,), jnp.float32),
        ],
    )(features, flat)


def kernel(features, coords):
    norm_coords, flat = _normalize(coords)
    flat2 = flat.reshape(_B, _N)
    vox = _scatter(features, flat2)
    return vox.reshape(_B, _C, _R, _R, _R), norm_coords
